# Initial kernel scaffold; baseline (speedup 1.0000x reference)
import functools

import jax
import jax.numpy as jnp
from jax import lax
from jax.experimental import pallas as pl
from jax.experimental.pallas import tpu as pltpu
from jax.experimental.pallas import tpu_sc as plsc

_LANES = 16
_IDX_CHUNK = 128


@functools.cache
def _make_pool(B, H, V, E):
    info = plsc.get_sparse_core_info()
    nc, ns = info.num_cores, info.num_subcores
    nw = nc * ns
    bpw = B // nw
    assert B % nw == 0 and E % _LANES == 0
    chunks = [(o, min(_IDX_CHUNK, H - o)) for o in range(0, H, _IDX_CHUNK)]
    mesh = plsc.VectorSubcoreMesh(core_axis_name="c", subcore_axis_name="s")

    def body(table_hbm, text_hbm, out_hbm, idx_v, rows_v, pooled_v, sem):
        wid = lax.axis_index("s") * nc + lax.axis_index("c")
        base = wid * bpw
        pltpu.sync_copy(text_hbm.at[pl.ds(base, bpw), :], idx_v)

        def row_step(i, carry):
            for (o, n) in chunks:
                pltpu.async_copy(
                    table_hbm.at[idx_v.at[i, pl.ds(o, n)]],
                    rows_v.at[pl.ds(o, n)], sem)
            for (o, n) in chunks:
                pltpu.make_async_copy(
                    table_hbm.at[idx_v.at[i, pl.ds(o, n)]],
                    rows_v.at[pl.ds(o, n)], sem).wait()

            def red(j, acc):
                return tuple(
                    acc[k] + rows_v[j, pl.ds(k * _LANES, _LANES)]
                    for k in range(E // _LANES))

            zeros = tuple(
                jnp.zeros((_LANES,), jnp.float32) for _ in range(E // _LANES))
            acc = lax.fori_loop(0, H, red, zeros)
            for k in range(E // _LANES):
                pooled_v[i, pl.ds(k * _LANES, _LANES)] = acc[k]
            return carry

        lax.fori_loop(0, bpw, row_step, 0)
        pltpu.sync_copy(pooled_v, out_hbm.at[pl.ds(base, bpw), :])

    return pl.kernel(
        body,
        out_type=jax.ShapeDtypeStruct((B, E), jnp.float32),
        mesh=mesh,
        scratch_types=[
            pltpu.VMEM((bpw, H), jnp.int32),
            pltpu.VMEM((H, E), jnp.float32),
            pltpu.VMEM((bpw, E), jnp.float32),
            pltpu.SemaphoreType.DMA,
        ],
    )


@functools.cache
def _make_dense(B, H, E, A, O):
    grid = 8
    bt = B // grid

    def body(p_ref, a_ref, w1_ref, w2_ref, b_ref, o_ref):
        t = p_ref[...] * (1.0 / H)
        a = a_ref[...]
        m = jnp.maximum(jnp.max(t, axis=1, keepdims=True),
                        jnp.max(a, axis=1, keepdims=True))
        et = jnp.exp(t - m)
        ea = jnp.exp(a - m)
        s = (jnp.sum(et, axis=1, keepdims=True)
             + jnp.sum(ea, axis=1, keepdims=True))
        acc = jnp.dot(et, w1_ref[...], preferred_element_type=jnp.float32)
        acc = acc + jnp.dot(ea, w2_ref[...], preferred_element_type=jnp.float32)
        o_ref[...] = acc / s + b_ref[...]

    return pl.pallas_call(
        body,
        grid=(grid,),
        in_specs=[
            pl.BlockSpec((bt, E), lambda i: (i, 0)),
            pl.BlockSpec((bt, A), lambda i: (i, 0)),
            pl.BlockSpec((E, O), lambda i: (0, 0)),
            pl.BlockSpec((A, O), lambda i: (0, 0)),
            pl.BlockSpec((1, O), lambda i: (0, 0)),
        ],
        out_specs=pl.BlockSpec((bt, O), lambda i: (i, 0)),
        out_shape=jax.ShapeDtypeStruct((B, O), jnp.float32),
    )


@jax.jit
def kernel(text, audio, table, W, b):
    B, H = text.shape
    V, E = table.shape
    A = audio.shape[1]
    O = W.shape[1]
    pooled = _make_pool(B, H, V, E)(table, text)
    return _make_dense(B, H, E, A, O)(
        pooled, audio, W[:E], W[E:], b.reshape(1, O))


# speedup vs baseline: 1.8842x; 1.8842x over previous
import functools

import jax
import jax.numpy as jnp
from jax import lax
from jax.experimental import pallas as pl
from jax.experimental.pallas import tpu as pltpu
from jax.experimental.pallas import tpu_sc as plsc

_LANES = 16
_IDX_CHUNK = 128


@functools.cache
def _make_pool(B, H, V, E):
    info = plsc.get_sparse_core_info()
    nc, ns = info.num_cores, info.num_subcores
    nw = nc * ns
    bpw = B // nw
    assert B % nw == 0 and E % _LANES == 0
    chunks = [(o, min(_IDX_CHUNK, H - o)) for o in range(0, H, _IDX_CHUNK)]
    mesh = plsc.VectorSubcoreMesh(core_axis_name="c", subcore_axis_name="s")

    def body(table_hbm, text_hbm, out_hbm, idx_v, rows_v, pooled_v, sem):
        wid = lax.axis_index("s") * nc + lax.axis_index("c")
        base = wid * bpw
        pltpu.sync_copy(text_hbm.at[pl.ds(base, bpw), :], idx_v)

        def row_step(i, carry):
            for (o, n) in chunks:
                pltpu.async_copy(
                    table_hbm.at[idx_v.at[i, pl.ds(o, n)]],
                    rows_v.at[pl.ds(o, n)], sem)
            for (o, n) in chunks:
                pltpu.make_async_copy(
                    table_hbm.at[idx_v.at[i, pl.ds(o, n)]],
                    rows_v.at[pl.ds(o, n)], sem).wait()

            def red(j, acc):
                return tuple(
                    acc[k] + rows_v[j, pl.ds(k * _LANES, _LANES)]
                    for k in range(E // _LANES))

            zeros = tuple(
                jnp.zeros((_LANES,), jnp.float32) for _ in range(E // _LANES))
            acc = lax.fori_loop(0, H, red, zeros)
            for k in range(E // _LANES):
                pooled_v[i, pl.ds(k * _LANES, _LANES)] = acc[k]
            return carry

        lax.fori_loop(0, bpw, row_step, 0)
        pltpu.sync_copy(pooled_v, out_hbm.at[pl.ds(base, bpw), :])

    return pl.kernel(
        body,
        out_type=jax.ShapeDtypeStruct((B, E), jnp.float32),
        mesh=mesh,
        compiler_params=pltpu.CompilerParams(use_tc_tiling_on_sc=False),
        scratch_types=[
            pltpu.VMEM((bpw, H), jnp.int32),
            pltpu.VMEM((H, E), jnp.float32),
            pltpu.VMEM((bpw, E), jnp.float32),
            pltpu.SemaphoreType.DMA,
        ],
    )


@functools.cache
def _make_dense(B, H, E, A, O):
    grid = 8
    bt = B // grid

    def body(p_ref, a_ref, w1_ref, w2_ref, b_ref, o_ref):
        t = p_ref[...] * (1.0 / H)
        a = a_ref[...]
        m = jnp.maximum(jnp.max(t, axis=1, keepdims=True),
                        jnp.max(a, axis=1, keepdims=True))
        et = jnp.exp(t - m)
        ea = jnp.exp(a - m)
        s = (jnp.sum(et, axis=1, keepdims=True)
             + jnp.sum(ea, axis=1, keepdims=True))
        acc = jnp.dot(et, w1_ref[...], preferred_element_type=jnp.float32)
        acc = acc + jnp.dot(ea, w2_ref[...], preferred_element_type=jnp.float32)
        o_ref[...] = acc / s + b_ref[...]

    return pl.pallas_call(
        body,
        grid=(grid,),
        in_specs=[
            pl.BlockSpec((bt, E), lambda i: (i, 0)),
            pl.BlockSpec((bt, A), lambda i: (i, 0)),
            pl.BlockSpec((E, O), lambda i: (0, 0)),
            pl.BlockSpec((A, O), lambda i: (0, 0)),
            pl.BlockSpec((1, O), lambda i: (0, 0)),
        ],
        out_specs=pl.BlockSpec((bt, O), lambda i: (i, 0)),
        out_shape=jax.ShapeDtypeStruct((B, O), jnp.float32),
    )


@jax.jit
def kernel(text, audio, table, W, b):
    B, H = text.shape
    V, E = table.shape
    A = audio.shape[1]
    O = W.shape[1]
    pooled = _make_pool(B, H, V, E)(table, text)
    return _make_dense(B, H, E, A, O)(
        pooled, audio, W[:E], W[E:], b.reshape(1, O))


# speedup vs baseline: 2.2816x; 1.2109x over previous
import functools

import jax
import jax.numpy as jnp
from jax import lax
from jax.experimental import pallas as pl
from jax.experimental.pallas import tpu as pltpu
from jax.experimental.pallas import tpu_sc as plsc

_LANES = 16
_IDX_CHUNK = 128


@functools.cache
def _make_pool(B, H, V, E):
    info = plsc.get_sparse_core_info()
    nc, ns = info.num_cores, info.num_subcores
    nw = nc * ns
    bpw = B // nw
    assert B % nw == 0 and E % _LANES == 0
    chunks = [(o, min(_IDX_CHUNK, H - o)) for o in range(0, H, _IDX_CHUNK)]
    mesh = plsc.VectorSubcoreMesh(core_axis_name="c", subcore_axis_name="s")

    ne = E // _LANES
    P = 8

    def body(table_hbm, text_hbm, out_hbm, idx_v, rows0_v, rows1_v,
             pooled_v, sem0, sem1):
        wid = lax.axis_index("s") * nc + lax.axis_index("c")
        base = wid * bpw
        pltpu.sync_copy(text_hbm.at[pl.ds(base, bpw), :], idx_v)

        def issue(i, buf, sem):
            for (o, n) in chunks:
                pltpu.async_copy(
                    table_hbm.at[idx_v.at[i, pl.ds(o, n)]],
                    buf.at[pl.ds(o, n)], sem)

        def drain(i, buf, sem):
            for (o, n) in chunks:
                pltpu.make_async_copy(
                    table_hbm.at[idx_v.at[i, pl.ds(o, n)]],
                    buf.at[pl.ds(o, n)], sem).wait()

        def reduce_into(buf, i):
            def red(jj, accs):
                out = []
                for p in range(P):
                    j = jj * P + p
                    out.append(tuple(
                        accs[p][k] + buf[j, pl.ds(k * _LANES, _LANES)]
                        for k in range(ne)))
                return tuple(out)

            zeros = tuple(
                tuple(jnp.zeros((_LANES,), jnp.float32) for _ in range(ne))
                for _ in range(P))
            accs = lax.fori_loop(0, H // P, red, zeros)
            rem = tuple(accs[0][k] for k in range(ne))
            for p in range(1, P):
                rem = tuple(rem[k] + accs[p][k] for k in range(ne))
            for j in range((H // P) * P, H):
                rem = tuple(rem[k] + buf[j, pl.ds(k * _LANES, _LANES)]
                            for k in range(ne))
            for k in range(ne):
                pooled_v[i, pl.ds(k * _LANES, _LANES)] = rem[k]

        issue(0, rows0_v, sem0)

        def pair_step(ii, carry):
            a = 2 * ii
            issue(a + 1, rows1_v, sem1)
            drain(a, rows0_v, sem0)
            reduce_into(rows0_v, a)
            issue(a + 2, rows0_v, sem0)
            drain(a + 1, rows1_v, sem1)
            reduce_into(rows1_v, a + 1)
            return carry

        lax.fori_loop(0, bpw // 2 - 1, pair_step, 0)
        a = bpw - 2
        issue(a + 1, rows1_v, sem1)
        drain(a, rows0_v, sem0)
        reduce_into(rows0_v, a)
        drain(a + 1, rows1_v, sem1)
        reduce_into(rows1_v, a + 1)

        pltpu.sync_copy(pooled_v, out_hbm.at[pl.ds(base, bpw), :])

    return pl.kernel(
        body,
        out_type=jax.ShapeDtypeStruct((B, E), jnp.float32),
        mesh=mesh,
        compiler_params=pltpu.CompilerParams(use_tc_tiling_on_sc=False),
        scratch_types=[
            pltpu.VMEM((bpw, H), jnp.int32),
            pltpu.VMEM((H, E), jnp.float32),
            pltpu.VMEM((H, E), jnp.float32),
            pltpu.VMEM((bpw, E), jnp.float32),
            pltpu.SemaphoreType.DMA,
            pltpu.SemaphoreType.DMA,
        ],
    )


@functools.cache
def _make_dense(B, H, E, A, O):
    grid = 8
    bt = B // grid

    def body(p_ref, a_ref, w1_ref, w2_ref, b_ref, o_ref):
        t = p_ref[...] * (1.0 / H)
        a = a_ref[...]
        m = jnp.maximum(jnp.max(t, axis=1, keepdims=True),
                        jnp.max(a, axis=1, keepdims=True))
        et = jnp.exp(t - m)
        ea = jnp.exp(a - m)
        s = (jnp.sum(et, axis=1, keepdims=True)
             + jnp.sum(ea, axis=1, keepdims=True))
        acc = jnp.dot(et, w1_ref[...], preferred_element_type=jnp.float32)
        acc = acc + jnp.dot(ea, w2_ref[...], preferred_element_type=jnp.float32)
        o_ref[...] = acc / s + b_ref[...]

    return pl.pallas_call(
        body,
        grid=(grid,),
        in_specs=[
            pl.BlockSpec((bt, E), lambda i: (i, 0)),
            pl.BlockSpec((bt, A), lambda i: (i, 0)),
            pl.BlockSpec((E, O), lambda i: (0, 0)),
            pl.BlockSpec((A, O), lambda i: (0, 0)),
            pl.BlockSpec((1, O), lambda i: (0, 0)),
        ],
        out_specs=pl.BlockSpec((bt, O), lambda i: (i, 0)),
        out_shape=jax.ShapeDtypeStruct((B, O), jnp.float32),
    )


@jax.jit
def kernel(text, audio, table, W, b):
    B, H = text.shape
    V, E = table.shape
    A = audio.shape[1]
    O = W.shape[1]
    pooled = _make_pool(B, H, V, E)(table, text)
    return _make_dense(B, H, E, A, O)(
        pooled, audio, W[:E], W[E:], b.reshape(1, O))
